# baseline (speedup 1.0000x reference)
import functools

import jax
import jax.numpy as jnp
from jax import lax
from jax.experimental import pallas as pl
from jax.experimental.pallas import tpu as pltpu
from jax.experimental.pallas import tpu_sc as plsc

_N = 10000
_F = 128
_FH = _F // 2
_E = 320000

_NP = 10240
_RPT = _NP // 16
_C = 128
_RB = 4
_IDXD = 6
_EP = 327680
_EPW = _EP // 16
_NOUT = _EPW // _C


def _mm_body(seq_ref, w_ref, a_ref, sf2_ref, fa_ref, mx_ref):
    sf = jnp.dot(seq_ref[...], w_ref[...], preferred_element_type=jnp.float32)
    sf2_ref[0] = sf[:, :_FH]
    sf2_ref[1] = sf[:, _FH:]
    fa = jnp.dot(sf, a_ref[...], preferred_element_type=jnp.float32)
    fa_ref[...] = fa
    mx_ref[...] = jnp.max(fa, axis=0, keepdims=True)


def _sc_edge_body(sfh, rowh, colh, f1h, f2h, mh, valsp, denomp,
                  f1b, f2b, mb, rowb, colb, exb, rowsb, vals_s, denom_s,
                  gsem, isem, ssem):
    c = lax.axis_index("c")
    s = lax.axis_index("s")
    ebase = s * _EPW
    zeros16 = jnp.zeros((16,), jnp.float32)

    def _idx_start(chunk, slot):
        base = ebase + chunk * _C
        pltpu.async_copy(rowh.at[pl.ds(base, _C)], rowb.at[slot, 0], isem)
        pltpu.async_copy(colh.at[pl.ds(base, _C)], colb.at[slot, 0], isem)

    def _idx_drain(chunk, slot):
        base = ebase + chunk * _C
        pltpu.make_async_copy(rowh.at[pl.ds(base, _C)],
                              rowb.at[slot, 0], isem).wait()
        pltpu.make_async_copy(colh.at[pl.ds(base, _C)],
                              colb.at[slot, 0], isem).wait()

    def _gather_start(slot, buf):
        pltpu.async_copy(sfh.at[c].at[colb.at[slot, 0]],
                         rowsb.at[pl.ds(buf * _C, _C), :], gsem)

    def _gather_drain(slot, buf):
        pltpu.make_async_copy(sfh.at[c].at[colb.at[slot, 0]],
                              rowsb.at[pl.ds(buf * _C, _C), :], gsem).wait()

    def _scatter_start(slot, buf):
        pltpu.async_copy(rowsb.at[pl.ds(buf * _C, _C), :],
                         vals_s.at[rowb.at[slot, 0]], ssem, add=True)
        pltpu.async_copy(exb.at[pl.ds(buf * _C, _C)],
                         denom_s.at[rowb.at[slot, 0]], ssem, add=True)

    def _scatter_drain(slot, buf):
        pltpu.make_async_copy(rowsb.at[pl.ds(buf * _C, _C), :],
                              vals_s.at[rowb.at[slot, 0]], ssem).wait()
        pltpu.make_async_copy(exb.at[pl.ds(buf * _C, _C)],
                              denom_s.at[rowb.at[slot, 0]], ssem).wait()

    _idx_start(0, 0)
    _idx_start(1, 1)
    _idx_start(2, 2)
    pltpu.sync_copy(f1h, f1b)
    pltpu.sync_copy(f2h, f2b)
    pltpu.sync_copy(mh, mb)
    def zrow(i, carry):
        for k in range(_FH // 16):
            rowsb[3 * _C + i, pl.ds(k * 16, 16)] = zeros16
        return carry
    lax.fori_loop(0, _C, zrow, 0, unroll=8)
    for q in range(_C // 16):
        exb[pl.ds(3 * _C + q * 16, 16)] = zeros16
    _idx_drain(0, 0)
    _gather_start(0, 0)
    _idx_drain(1, 1)
    _gather_start(1, 1)
    for z in range(_RPT // _C):
        pltpu.sync_copy(rowsb.at[pl.ds(3 * _C, _C), :],
                        vals_s.at[pl.ds(s * _RPT + z * _C, _C), :])
        pltpu.sync_copy(exb.at[pl.ds(3 * _C, _C)],
                        denom_s.at[pl.ds(s * _RPT + z * _C, _C)])
    plsc.subcore_barrier()
    mv = mb[...]

    def step(g, carry):
        j0 = lax.rem(g, _IDXD)
        j2 = lax.rem(g + 2, _IDXD)
        j3 = lax.rem(g + 3, _IDXD)
        jd = lax.rem(g + _IDXD - 2, _IDXD)
        b0 = lax.rem(g, _RB)
        b2 = lax.rem(g + 2, _RB)
        bd = lax.rem(g + _RB - 2, _RB)

        @pl.when(g >= 2)
        def _():
            _scatter_drain(jd, bd)

        @pl.when(g + 3 < _NOUT)
        def _():
            _idx_start(g + 3, j3)

        eoff = b0 * _C
        _gather_drain(j0, b0)

        @pl.when(g + 2 < _NOUT)
        def _():
            _idx_drain(g + 2, j2)
            _gather_start(j2, b2)

        def group(q, carry2):
            qb = q * 16
            rv = rowb[j0, 0, pl.ds(qb, 16)]
            cv = colb[j0, 0, pl.ds(qb, 16)]
            x = plsc.load_gather(f1b, [rv]) + plsc.load_gather(f2b, [cv])
            lr = jnp.maximum(x, 0.2 * x)
            ex16 = jnp.exp(lr - mv)
            exb[pl.ds(eoff + qb, 16)] = ex16
            for j in range(16):
                ev = jnp.broadcast_to(ex16[j], (16,))
                for k in range(_FH // 16):
                    rowsb[eoff + qb + j, pl.ds(k * 16, 16)] = (
                        rowsb[eoff + qb + j, pl.ds(k * 16, 16)] * ev)
            return carry2
        lax.fori_loop(0, _C // 16, group, 0)

        _scatter_start(j0, b0)
        return carry
    lax.fori_loop(0, _NOUT, step, 0)
    _scatter_drain(lax.rem(jnp.int32(_NOUT - 2), _IDXD),
                   lax.rem(jnp.int32(_NOUT - 2), _RB))
    _scatter_drain(lax.rem(jnp.int32(_NOUT - 1), _IDXD),
                   lax.rem(jnp.int32(_NOUT - 1), _RB))

    plsc.subcore_barrier()
    pltpu.sync_copy(vals_s.at[pl.ds(s * _RPT, _RPT), :],
                    valsp.at[c, pl.ds(s * _RPT, _RPT), :])

    @pl.when(c == 0)
    def _():
        pltpu.sync_copy(denom_s.at[pl.ds(s * _RPT, _RPT)],
                        denomp.at[pl.ds(s * _RPT, _RPT)])


_sc_edge = functools.partial(
    pl.kernel,
    out_type=[jax.ShapeDtypeStruct((2, _NP, _FH), jnp.float32),
              jax.ShapeDtypeStruct((_NP,), jnp.float32)],
    mesh=plsc.VectorSubcoreMesh(core_axis_name="c", subcore_axis_name="s"),
    compiler_params=pltpu.CompilerParams(needs_layout_passes=False,
                                         use_tc_tiling_on_sc=False),
    scratch_types=[
        pltpu.VMEM((_N,), jnp.float32),
        pltpu.VMEM((_N,), jnp.float32),
        pltpu.VMEM((16,), jnp.float32),
        pltpu.VMEM((_IDXD, 1, _C), jnp.int32),
        pltpu.VMEM((_IDXD, 1, _C), jnp.int32),
        pltpu.VMEM((_RB * _C,), jnp.float32),
        pltpu.VMEM((_RB * _C, _FH), jnp.float32),
        pltpu.VMEM_SHARED((_NP, _FH), jnp.float32),
        pltpu.VMEM_SHARED((_NP,), jnp.float32),
        pltpu.SemaphoreType.DMA,
        pltpu.SemaphoreType.DMA,
        pltpu.SemaphoreType.DMA,
    ],
)(_sc_edge_body)


def _fin_body(v_ref, d_ref, b_ref, o_ref):
    v = jnp.concatenate([v_ref[0], v_ref[1]], axis=-1)
    d = (d_ref[0] + 1e-16)[:, None]
    o_ref[...] = jnp.maximum(v / d + b_ref[...], 0.0)


def kernel(seq, W, a1, b1, a2, b2, bias, edge_index):
    n, f = seq.shape
    seq = seq.astype(jnp.float32)
    A = jnp.zeros((f, _F), jnp.float32)
    A = A.at[:, 0].set(a1[:, 0]).at[:, 1].set(a2[:, 0])

    sf2, fa, mx = pl.pallas_call(
        _mm_body,
        out_shape=[jax.ShapeDtypeStruct((2, n, _FH), jnp.float32),
                   jax.ShapeDtypeStruct((n, _F), jnp.float32),
                   jax.ShapeDtypeStruct((1, _F), jnp.float32)],
    )(seq, W.astype(jnp.float32), A)

    f1 = fa[:, 0] + b1[0]
    f2 = fa[:, 1] + b2[0]
    mval = mx[0, 0] + mx[0, 1] + b1[0] + b2[0]
    m = jnp.maximum(mval, 0.2 * mval)
    marr = jnp.full((16,), m, jnp.float32)

    pad = _EP - _E
    rowp = jnp.concatenate(
        [edge_index[0],
         _N + (jnp.arange(pad, dtype=jnp.int32) % (_NP - _N))])
    colp = jnp.concatenate([edge_index[1], jnp.zeros((pad,), jnp.int32)])
    valsp, denomp = _sc_edge(sf2, rowp, colp, f1, f2, marr)

    blk = 1024
    out = pl.pallas_call(
        _fin_body,
        grid=(_NP // blk,),
        in_specs=[pl.BlockSpec((2, blk, _FH), lambda i: (0, i, 0)),
                  pl.BlockSpec((1, blk), lambda i: (0, i)),
                  pl.BlockSpec((1, _F), lambda i: (0, 0))],
        out_specs=pl.BlockSpec((blk, _F), lambda i: (i, 0)),
        out_shape=jax.ShapeDtypeStruct((_NP, _F), jnp.float32),
    )(valsp, denomp.reshape(1, _NP), bias.reshape(1, _F))
    return out[:n]


# speedup vs baseline: 1.3575x; 1.3575x over previous
import functools

import jax
import jax.numpy as jnp
from jax import lax
from jax.experimental import pallas as pl
from jax.experimental.pallas import tpu as pltpu
from jax.experimental.pallas import tpu_sc as plsc

_N = 10000
_F = 128
_FH = _F // 2
_E = 320000

_NP = 10240
_RPT = _NP // 16
_C = 128
_RB = 4
_IDXD = 6
_EP = 327680
_EPW = _EP // 16
_NOUT = _EPW // _C


def _mm_body(seq_ref, w_ref, a_ref, sf2_ref, fa_ref, mx_ref):
    sf = jnp.dot(seq_ref[...], w_ref[...], preferred_element_type=jnp.float32)
    sf2_ref[0] = sf[:, :_FH]
    sf2_ref[1] = sf[:, _FH:]
    fa = jnp.dot(sf, a_ref[...], preferred_element_type=jnp.float32)
    fa_ref[...] = fa
    mx_ref[...] = jnp.max(fa, axis=0, keepdims=True)


def _sc_edge_body(sfh, rowh, colh, f1h, f2h, mh, valsp, denomp,
                  f1b, f2b, mb, rowb, colb, exb, rowsb, vals_s, denom_s,
                  gsem, isem, ssem):
    c = lax.axis_index("c")
    s = lax.axis_index("s")
    ebase = s * _EPW
    zeros16 = jnp.zeros((16,), jnp.float32)

    def _idx_start(chunk, slot):
        base = ebase + chunk * _C
        pltpu.async_copy(rowh.at[pl.ds(base, _C)], rowb.at[slot, 0], isem)
        pltpu.async_copy(colh.at[pl.ds(base, _C)], colb.at[slot, 0], isem)

    def _idx_drain(chunk, slot):
        base = ebase + chunk * _C
        pltpu.make_async_copy(rowh.at[pl.ds(base, _C)],
                              rowb.at[slot, 0], isem).wait()
        pltpu.make_async_copy(colh.at[pl.ds(base, _C)],
                              colb.at[slot, 0], isem).wait()

    def _gather_start(slot, buf):
        pltpu.async_copy(sfh.at[c].at[colb.at[slot, 0]],
                         rowsb.at[pl.ds(buf * _C, _C), :], gsem)

    def _gather_drain(slot, buf):
        pltpu.make_async_copy(sfh.at[c].at[colb.at[slot, 0]],
                              rowsb.at[pl.ds(buf * _C, _C), :], gsem).wait()

    def _scatter_start(slot, buf):
        pltpu.async_copy(rowsb.at[pl.ds(buf * _C, _C), :],
                         vals_s.at[rowb.at[slot, 0]], ssem, add=True)
        pltpu.async_copy(exb.at[pl.ds(buf * _C, _C)],
                         denom_s.at[rowb.at[slot, 0]], ssem, add=True)

    def _scatter_drain(slot, buf):
        pltpu.make_async_copy(rowsb.at[pl.ds(buf * _C, _C), :],
                              vals_s.at[rowb.at[slot, 0]], ssem).wait()
        pltpu.make_async_copy(exb.at[pl.ds(buf * _C, _C)],
                              denom_s.at[rowb.at[slot, 0]], ssem).wait()

    _idx_start(0, 0)
    _idx_start(1, 1)
    _idx_start(2, 2)
    pltpu.sync_copy(f1h, f1b)
    pltpu.sync_copy(f2h, f2b)
    pltpu.sync_copy(mh, mb)
    def zrow(i, carry):
        for k in range(_FH // 16):
            rowsb[3 * _C + i, pl.ds(k * 16, 16)] = zeros16
        return carry
    lax.fori_loop(0, _C, zrow, 0, unroll=8)
    for q in range(_C // 16):
        exb[pl.ds(3 * _C + q * 16, 16)] = zeros16
    _idx_drain(0, 0)
    _gather_start(0, 0)
    _idx_drain(1, 1)
    _gather_start(1, 1)
    for z in range(_RPT // _C):
        pltpu.sync_copy(rowsb.at[pl.ds(3 * _C, _C), :],
                        vals_s.at[pl.ds(s * _RPT + z * _C, _C), :])
        pltpu.sync_copy(exb.at[pl.ds(3 * _C, _C)],
                        denom_s.at[pl.ds(s * _RPT + z * _C, _C)])
    plsc.subcore_barrier()
    mv = mb[...]

    def step(g, carry):
        j0 = lax.rem(g, _IDXD)
        j2 = lax.rem(g + 2, _IDXD)
        j3 = lax.rem(g + 3, _IDXD)
        jd = lax.rem(g + _IDXD - 2, _IDXD)
        b0 = lax.rem(g, _RB)
        b2 = lax.rem(g + 2, _RB)
        bd = lax.rem(g + _RB - 2, _RB)

        @pl.when(g >= 2)
        def _():
            _scatter_drain(jd, bd)

        @pl.when(g + 3 < _NOUT)
        def _():
            _idx_start(g + 3, j3)

        eoff = b0 * _C
        _gather_drain(j0, b0)

        @pl.when(g + 2 < _NOUT)
        def _():
            _idx_drain(g + 2, j2)
            _gather_start(j2, b2)

        def group(q, carry2):
            qb = q * 16
            rv = rowb[j0, 0, pl.ds(qb, 16)]
            cv = colb[j0, 0, pl.ds(qb, 16)]
            x = plsc.load_gather(f1b, [rv]) + plsc.load_gather(f2b, [cv])
            lr = jnp.maximum(x, 0.2 * x)
            ex16 = jnp.exp(lr - mv)
            exb[pl.ds(eoff + qb, 16)] = ex16
            for j in range(16):
                ev = jnp.broadcast_to(ex16[j], (16,))
                for k in range(_FH // 16):
                    rowsb[eoff + qb + j, pl.ds(k * 16, 16)] = (
                        rowsb[eoff + qb + j, pl.ds(k * 16, 16)] * ev)
            return carry2
        lax.fori_loop(0, _C // 16, group, 0, unroll=8)

        _scatter_start(j0, b0)
        return carry
    lax.fori_loop(0, _NOUT, step, 0)
    _scatter_drain(lax.rem(jnp.int32(_NOUT - 2), _IDXD),
                   lax.rem(jnp.int32(_NOUT - 2), _RB))
    _scatter_drain(lax.rem(jnp.int32(_NOUT - 1), _IDXD),
                   lax.rem(jnp.int32(_NOUT - 1), _RB))

    plsc.subcore_barrier()
    pltpu.sync_copy(vals_s.at[pl.ds(s * _RPT, _RPT), :],
                    valsp.at[c, pl.ds(s * _RPT, _RPT), :])

    @pl.when(c == 0)
    def _():
        pltpu.sync_copy(denom_s.at[pl.ds(s * _RPT, _RPT)],
                        denomp.at[pl.ds(s * _RPT, _RPT)])


_sc_edge = functools.partial(
    pl.kernel,
    out_type=[jax.ShapeDtypeStruct((2, _NP, _FH), jnp.float32),
              jax.ShapeDtypeStruct((_NP,), jnp.float32)],
    mesh=plsc.VectorSubcoreMesh(core_axis_name="c", subcore_axis_name="s"),
    compiler_params=pltpu.CompilerParams(needs_layout_passes=False,
                                         use_tc_tiling_on_sc=False),
    scratch_types=[
        pltpu.VMEM((_N,), jnp.float32),
        pltpu.VMEM((_N,), jnp.float32),
        pltpu.VMEM((16,), jnp.float32),
        pltpu.VMEM((_IDXD, 1, _C), jnp.int32),
        pltpu.VMEM((_IDXD, 1, _C), jnp.int32),
        pltpu.VMEM((_RB * _C,), jnp.float32),
        pltpu.VMEM((_RB * _C, _FH), jnp.float32),
        pltpu.VMEM_SHARED((_NP, _FH), jnp.float32),
        pltpu.VMEM_SHARED((_NP,), jnp.float32),
        pltpu.SemaphoreType.DMA,
        pltpu.SemaphoreType.DMA,
        pltpu.SemaphoreType.DMA,
    ],
)(_sc_edge_body)


def _fin_body(v_ref, d_ref, b_ref, o_ref):
    v = jnp.concatenate([v_ref[0], v_ref[1]], axis=-1)
    d = (d_ref[0] + 1e-16)[:, None]
    o_ref[...] = jnp.maximum(v / d + b_ref[...], 0.0)


def kernel(seq, W, a1, b1, a2, b2, bias, edge_index):
    n, f = seq.shape
    seq = seq.astype(jnp.float32)
    A = jnp.zeros((f, _F), jnp.float32)
    A = A.at[:, 0].set(a1[:, 0]).at[:, 1].set(a2[:, 0])

    sf2, fa, mx = pl.pallas_call(
        _mm_body,
        out_shape=[jax.ShapeDtypeStruct((2, n, _FH), jnp.float32),
                   jax.ShapeDtypeStruct((n, _F), jnp.float32),
                   jax.ShapeDtypeStruct((1, _F), jnp.float32)],
    )(seq, W.astype(jnp.float32), A)

    f1 = fa[:, 0] + b1[0]
    f2 = fa[:, 1] + b2[0]
    mval = mx[0, 0] + mx[0, 1] + b1[0] + b2[0]
    m = jnp.maximum(mval, 0.2 * mval)
    marr = jnp.full((16,), m, jnp.float32)

    pad = _EP - _E
    rowp = jnp.concatenate(
        [edge_index[0],
         _N + (jnp.arange(pad, dtype=jnp.int32) % (_NP - _N))])
    colp = jnp.concatenate([edge_index[1], jnp.zeros((pad,), jnp.int32)])
    valsp, denomp = _sc_edge(sf2, rowp, colp, f1, f2, marr)

    blk = 1024
    out = pl.pallas_call(
        _fin_body,
        grid=(_NP // blk,),
        in_specs=[pl.BlockSpec((2, blk, _FH), lambda i: (0, i, 0)),
                  pl.BlockSpec((1, blk), lambda i: (0, i)),
                  pl.BlockSpec((1, _F), lambda i: (0, 0))],
        out_specs=pl.BlockSpec((blk, _F), lambda i: (i, 0)),
        out_shape=jax.ShapeDtypeStruct((_NP, _F), jnp.float32),
    )(valsp, denomp.reshape(1, _NP), bias.reshape(1, _F))
    return out[:n]
